# baseline (speedup 1.0000x reference)
import functools

import jax
import jax.numpy as jnp
from jax import lax
from jax.experimental import pallas as pl
from jax.experimental.pallas import tpu as pltpu
from jax.experimental.pallas import tpu_sc as plsc

B = 4096
L = 200
D = 32
NC = 2
NS = 16
NW = NC * NS
BPW = B // NW
GATHER_CHUNKS = ((0, 104), (104, 96))


def _sc_embed_body(x_hbm, table_hbm, out_hbm, idx_v, rows_v, out_v, gsem):
    wid = lax.axis_index("s") * NC + lax.axis_index("c")
    b0 = wid * BPW

    pltpu.sync_copy(x_hbm.at[pl.ds(b0, BPW)], idx_v)

    lane = lax.iota(jnp.int32, 16)
    lane_hi = lane + 16

    def batch_body(brel, carry):
        cps = [
            pltpu.async_copy(
                table_hbm.at[idx_v.at[brel, pl.ds(off, n)]],
                rows_v.at[pl.ds(off, n)],
                gsem,
            )
            for off, n in GATHER_CHUNKS
        ]
        for cp in cps:
            cp.wait()

        def tbody(l, c):
            v0 = rows_v[l, pl.ds(0, 16)]
            v1 = rows_v[l, pl.ds(16, 16)]
            lv = jnp.full((16,), l, jnp.int32)
            plsc.store_scatter(out_v, [lane, lv], v0)
            plsc.store_scatter(out_v, [lane_hi, lv], v1)
            return c

        lax.fori_loop(0, L, tbody, 0, unroll=4)

        pltpu.sync_copy(out_v, out_hbm.at[b0 + brel])
        return carry

    lax.fori_loop(0, BPW, batch_body, 0)


@jax.jit
def _embed(x, table):
    mesh = plsc.VectorSubcoreMesh(
        core_axis_name="c", subcore_axis_name="s", num_cores=NC, num_subcores=NS
    )
    return pl.kernel(
        _sc_embed_body,
        out_type=jax.ShapeDtypeStruct((B, D, L), jnp.float32),
        mesh=mesh,
        scratch_types=[
            pltpu.VMEM((BPW, L), jnp.int32),
            pltpu.VMEM((L, D), jnp.float32),
            pltpu.VMEM((D, L), jnp.float32),
            pltpu.SemaphoreType.DMA,
        ],
        compiler_params=pltpu.CompilerParams(
            needs_layout_passes=False, use_tc_tiling_on_sc=False
        ),
    )(x, table)


def kernel(x, table):
    return _embed(x.astype(jnp.int32), table)


# speedup vs baseline: 1.0701x; 1.0701x over previous
import functools

import jax
import jax.numpy as jnp
from jax import lax
from jax.experimental import pallas as pl
from jax.experimental.pallas import tpu as pltpu
from jax.experimental.pallas import tpu_sc as plsc

B = 4096
L = 200
D = 32
NC = 2
NS = 16
NW = NC * NS
LT = L // 8
NSLAB = L // 4
SLAB_ROWS = 4 * 128


def _sc_embed_body(x4_hbm, table_hbm, out5_hbm, idx_v, rows_v, out_v,
                   gsem0, gsem1, wsem0, wsem1):
    w = lax.axis_index("s") * NC + lax.axis_index("c")

    def idx_body(lt, carry):
        pltpu.sync_copy(x4_hbm.at[lt, w], idx_v.at[lt])
        return carry

    lax.fori_loop(0, LT, idx_body, 0)

    lane = lax.iota(jnp.int32, 16)
    lane_hi = lane + 16
    gsems = (gsem0, gsem1)
    wsems = (wsem0, wsem1)

    def gather_cps(s, j):
        lt = s // 2
        li0 = (s % 2) * 4
        return [
            pltpu.make_async_copy(
                table_hbm.at[idx_v.at[lt, li0 + k]],
                rows_v.at[j].at[pl.ds(k * 128, 128)],
                gsems[j],
            )
            for k in range(4)
        ]

    def write_cp(s, j):
        return pltpu.make_async_copy(
            out_v.at[j],
            out5_hbm.at[:, s // 2, w, pl.ds((s % 2) * 4, 4)],
            wsems[j],
        )

    for cp in gather_cps(0, 0) + gather_cps(1, 1):
        cp.start()

    def pair_body(g, carry):
        for j in (0, 1):
            s = 2 * g + j
            for cp in gather_cps(s, j):
                cp.wait()

            @pl.when(s >= 2)
            def _():
                write_cp(s - 2, j).wait()

            for k in range(4):
                def tbody(bi, c, k=k):
                    v0 = rows_v[j, k * 128 + bi, pl.ds(0, 16)]
                    v1 = rows_v[j, k * 128 + bi, pl.ds(16, 16)]
                    kf = jnp.full((16,), k, jnp.int32)
                    bf = jnp.full((16,), bi, jnp.int32)
                    plsc.store_scatter(out_v.at[j], [lane, kf, bf], v0)
                    plsc.store_scatter(out_v.at[j], [lane_hi, kf, bf], v1)
                    return c

                lax.fori_loop(0, 128, tbody, 0, unroll=4)

            write_cp(s, j).start()

            @pl.when(s + 2 < NSLAB)
            def _():
                for cp in gather_cps(s + 2, j):
                    cp.start()
        return carry

    lax.fori_loop(0, NSLAB // 2, pair_body, 0)
    write_cp(NSLAB - 2, 0).wait()
    write_cp(NSLAB - 1, 1).wait()


@jax.jit
def _embed(x4, table):
    mesh = plsc.VectorSubcoreMesh(
        core_axis_name="c", subcore_axis_name="s", num_cores=NC, num_subcores=NS
    )
    return pl.kernel(
        _sc_embed_body,
        out_type=jax.ShapeDtypeStruct((D, LT, NW, 8, 128), jnp.float32),
        mesh=mesh,
        scratch_types=[
            pltpu.VMEM((LT, 8, 128), jnp.int32),
            pltpu.VMEM((2, SLAB_ROWS, D), jnp.float32),
            pltpu.VMEM((2, D, 4, 128), jnp.float32),
            pltpu.SemaphoreType.DMA,
            pltpu.SemaphoreType.DMA,
            pltpu.SemaphoreType.DMA,
            pltpu.SemaphoreType.DMA,
        ],
        compiler_params=pltpu.CompilerParams(
            needs_layout_passes=False, use_tc_tiling_on_sc=False
        ),
    )(x4, table)


def kernel(x, table):
    x4 = x.astype(jnp.int32).T.reshape(LT, 8, NW, 128).transpose(0, 2, 1, 3)
    out5 = _embed(x4, table)
    return out5.transpose(2, 4, 0, 1, 3).reshape(B, D, L)


# speedup vs baseline: 1.0839x; 1.0129x over previous
import functools

import jax
import jax.numpy as jnp
from jax import lax
from jax.experimental import pallas as pl
from jax.experimental.pallas import tpu as pltpu
from jax.experimental.pallas import tpu_sc as plsc

B = 4096
L = 200
D = 32
NC = 2
NS = 16
NW = NC * NS
LT = L // 8
NSLAB = L // 4
SLAB_ROWS = 4 * 128


def _sc_embed_body(x4_hbm, table_hbm, out5_hbm, idx_v, rows_v, out_v,
                   gsem0, gsem1, wsem0, wsem1):
    w = lax.axis_index("s") * NC + lax.axis_index("c")

    def idx_cp(lt):
        return pltpu.make_async_copy(x4_hbm.at[lt, w], idx_v.at[lt], wsem0)

    def idx_fire(lt, carry):
        idx_cp(lt).start()
        return carry

    def idx_drain(lt, carry):
        idx_cp(lt).wait()
        return carry

    lax.fori_loop(0, LT, idx_fire, 0)
    lax.fori_loop(0, LT, idx_drain, 0)

    lane = lax.iota(jnp.int32, 16)
    lane_hi = lane + 16
    gsems = (gsem0, gsem1)
    wsems = (wsem0, wsem1)

    def gather_cps(s, j):
        lt = s // 2
        li0 = (s % 2) * 4
        return [
            pltpu.make_async_copy(
                table_hbm.at[idx_v.at[lt, li0 + k]],
                rows_v.at[j].at[pl.ds(k * 128, 128)],
                gsems[j],
            )
            for k in range(4)
        ]

    def write_cp(s, j):
        return pltpu.make_async_copy(
            out_v.at[j],
            out5_hbm.at[:, s // 2, w, pl.ds((s % 2) * 512, 512)],
            wsems[j],
        )

    for cp in gather_cps(0, 0) + gather_cps(1, 1):
        cp.start()

    def pair_body(g, carry):
        for j in (0, 1):
            s = 2 * g + j
            for cp in gather_cps(s, j):
                cp.wait()

            @pl.when(s >= 2)
            def _():
                write_cp(s - 2, j).wait()

            def tbody(r, c):
                v0 = rows_v[j, r, pl.ds(0, 16)]
                v1 = rows_v[j, r, pl.ds(16, 16)]
                rf = jnp.full((16,), r, jnp.int32)
                plsc.store_scatter(out_v.at[j], [lane, rf], v0)
                plsc.store_scatter(out_v.at[j], [lane_hi, rf], v1)
                return c

            lax.fori_loop(0, SLAB_ROWS, tbody, 0, unroll=8)

            write_cp(s, j).start()

            @pl.when(s + 2 < NSLAB)
            def _():
                for cp in gather_cps(s + 2, j):
                    cp.start()
        return carry

    lax.fori_loop(0, NSLAB // 2, pair_body, 0)
    write_cp(NSLAB - 2, 0).wait()
    write_cp(NSLAB - 1, 1).wait()


@jax.jit
def _embed(x4, table):
    mesh = plsc.VectorSubcoreMesh(
        core_axis_name="c", subcore_axis_name="s", num_cores=NC, num_subcores=NS
    )
    return pl.kernel(
        _sc_embed_body,
        out_type=jax.ShapeDtypeStruct((D, LT, NW, 1024), jnp.float32),
        mesh=mesh,
        scratch_types=[
            pltpu.VMEM((LT, 8, 128), jnp.int32),
            pltpu.VMEM((2, SLAB_ROWS, D), jnp.float32),
            pltpu.VMEM((2, D, 512), jnp.float32),
            pltpu.SemaphoreType.DMA,
            pltpu.SemaphoreType.DMA,
            pltpu.SemaphoreType.DMA,
            pltpu.SemaphoreType.DMA,
        ],
        compiler_params=pltpu.CompilerParams(
            needs_layout_passes=False, use_tc_tiling_on_sc=False
        ),
    )(x4, table)


def kernel(x, table):
    x4 = x.astype(jnp.int32).T.reshape(LT, 8, NW, 128).transpose(0, 2, 1, 3)
    out5 = _embed(x4, table)
    return (
        out5.reshape(D, LT, NW, 8, 128)
        .transpose(2, 4, 0, 1, 3)
        .reshape(B, D, L)
    )
